# baseline (speedup 1.0000x reference)
import functools

import numpy as np
import jax
import jax.numpy as jnp
from jax import lax
from jax.experimental import pallas as pl
from jax.experimental.pallas import tpu as pltpu
from jax.experimental.pallas import tpu_sc as plsc

_N = 4
_ROWS = 16384
_COLS = 2 ** _N
_NC = 1
_NW = _NC * 16
_RPW = _ROWS // _NW
_TILE = 16
_TILES = _RPW // _TILE

_bits = ((np.arange(_COLS)[:, None] & (2 ** np.arange(_N - 1, -1, -1))) != 0)
_SIGNS = [np.where(_bits[:, 2 * i] == _bits[:, 2 * i + 1], 1, -1)
          for i in range(_N // 2)]

_mesh = plsc.VectorSubcoreMesh(
    core_axis_name="c", subcore_axis_name="s", num_cores=_NC)


@functools.partial(
    pl.kernel,
    mesh=_mesh,
    out_type=jax.ShapeDtypeStruct((2, _ROWS), jnp.float32),
    scratch_types=[
        pltpu.VMEM((_COLS, _RPW), jnp.float32),
        pltpu.VMEM((2, _RPW), jnp.float32),
    ],
    compiler_params=pltpu.CompilerParams(
        needs_layout_passes=False,
        skip_device_barrier=True,
        disable_bounds_checks=True,
        disable_semaphore_checks=True,
    ),
)
def _decode_sc(xt_hbm, out_hbm, xv, ov):
    wid = lax.axis_index("s") * _NC + lax.axis_index("c")
    base = wid * _RPW
    pltpu.sync_copy(xt_hbm.at[:, pl.ds(base, _RPW)], xv)

    def body(t, carry):
        off = pl.multiple_of(t * _TILE, _TILE)
        acc = [None, None]
        for j in range(_COLS):
            col = xv[j, pl.ds(off, _TILE)]
            for i in range(2):
                if acc[i] is None:
                    acc[i] = col if _SIGNS[i][j] > 0 else -col
                elif _SIGNS[i][j] > 0:
                    acc[i] = acc[i] + col
                else:
                    acc[i] = acc[i] - col
        ov[0, pl.ds(off, _TILE)] = acc[0]
        ov[1, pl.ds(off, _TILE)] = acc[1]
        return carry

    lax.fori_loop(0, _TILES, body, 0, unroll=4)

    pltpu.sync_copy(ov, out_hbm.at[:, pl.ds(base, _RPW)])


def kernel(input):
    return _decode_sc(input.T).T


# speedup vs baseline: 1.0325x; 1.0325x over previous
import functools

import numpy as np
import jax
import jax.numpy as jnp
from jax import lax
from jax.experimental import pallas as pl
from jax.experimental.pallas import tpu as pltpu
from jax.experimental.pallas import tpu_sc as plsc

_N = 4
_ROWS = 16384
_COLS = 2 ** _N
_NC = 1
_NW = _NC * 16
_RPW = _ROWS // _NW
_TILE = 16
_TILES = _RPW // _TILE

_bits = ((np.arange(_COLS)[:, None] & (2 ** np.arange(_N - 1, -1, -1))) != 0)
_SIGNS = [np.where(_bits[:, 2 * i] == _bits[:, 2 * i + 1], 1, -1)
          for i in range(_N // 2)]

_mesh = plsc.VectorSubcoreMesh(
    core_axis_name="c", subcore_axis_name="s", num_cores=_NC)


@functools.partial(
    pl.kernel,
    mesh=_mesh,
    out_type=jax.ShapeDtypeStruct((2, _ROWS), jnp.float32),
    scratch_types=[
        pltpu.VMEM((_COLS, _RPW), jnp.float32),
        pltpu.VMEM((2, _RPW), jnp.float32),
    ],
    compiler_params=pltpu.CompilerParams(
        needs_layout_passes=False,
        skip_device_barrier=True,
        disable_bounds_checks=True,
        disable_semaphore_checks=True,
    ),
)
def _decode_sc(xt_hbm, out_hbm, xv, ov):
    wid = lax.axis_index("s") * _NC + lax.axis_index("c")
    base = wid * _RPW
    pltpu.sync_copy(xt_hbm.at[:, pl.ds(base, _RPW)], xv)

    def body(t, carry):
        off = pl.multiple_of(t * _TILE, _TILE)
        acc = [None, None]
        for j in range(_COLS):
            col = xv[j, pl.ds(off, _TILE)]
            for i in range(2):
                if acc[i] is None:
                    acc[i] = col if _SIGNS[i][j] > 0 else -col
                elif _SIGNS[i][j] > 0:
                    acc[i] = acc[i] + col
                else:
                    acc[i] = acc[i] - col
        ov[0, pl.ds(off, _TILE)] = acc[0]
        ov[1, pl.ds(off, _TILE)] = acc[1]
        return carry

    lax.fori_loop(0, _TILES, body, 0)

    pltpu.sync_copy(ov, out_hbm.at[:, pl.ds(base, _RPW)])


def kernel(input):
    return _decode_sc(input.T).T
